# baseline (speedup 1.0000x reference)
import functools

import jax
import jax.numpy as jnp
from jax import lax
from jax.experimental import pallas as pl
from jax.experimental.pallas import tpu as pltpu
from jax.experimental.pallas import tpu_sc as plsc

_INFO = plsc.get_sparse_core_info()
_NC, _NS = _INFO.num_cores, _INFO.num_subcores
_NW = _NC * _NS
_PPB = 56
_LP = 128
_L = 16


@functools.partial(jax.jit, static_argnames=("nbatch", "npb", "d", "sb"))
def _sc_gather(table_p, idxf, *, nbatch, npb, d, sb):
    bat_w = nbatch // _NW
    nsb = bat_w // sb
    mesh = plsc.VectorSubcoreMesh(core_axis_name="c", subcore_axis_name="s")

    @functools.partial(
        pl.kernel,
        mesh=mesh,
        out_type=jax.ShapeDtypeStruct((nbatch, npb, d), jnp.float32),
        compiler_params=pltpu.CompilerParams(use_tc_tiling_on_sc=True),
        scratch_types=[
            pltpu.VMEM((bat_w * _PPB,), jnp.int32),
            pltpu.VMEM((2, sb, _PPB, _LP), jnp.float32),
            pltpu.VMEM((2, sb, npb, d), jnp.float32),
            [pltpu.SemaphoreType.DMA] * 2,
            [pltpu.SemaphoreType.DMA] * 2,
        ],
    )
    def k(table_hbm, idx_hbm, out_hbm, idx_v, slots_v, stage_v, gsem, osem):
        wid = lax.axis_index("s") * _NC + lax.axis_index("c")
        bbase = wid * bat_w
        pltpu.sync_copy(idx_hbm.at[pl.ds(bbase * _PPB, bat_w * _PPB)], idx_v)

        def gather_copy(s, p, i):
            return pltpu.make_async_copy(
                table_hbm.at[idx_v.at[pl.ds((s * sb + i) * _PPB, _PPB)]],
                slots_v.at[p].at[i],
                gsem[p],
            )

        def extract(p):
            def row(r, carry):
                for i in range(sb):
                    for c in range(d // _L):
                        stage_v[p, i, r, pl.ds(c * _L, _L)] = slots_v[
                            p, i, r, pl.ds(c * _L, _L)
                        ]
                return carry

            lax.fori_loop(0, npb, row, 0)

        def out_copy(s, p):
            return pltpu.make_async_copy(
                stage_v.at[p],
                out_hbm.at[pl.ds(bbase + s * sb, sb)],
                osem[p],
            )

        for i in range(sb):
            gather_copy(0, 0, i).start()
        for i in range(sb):
            gather_copy(1, 1, i).start()

        def group(g, carry):
            for p in range(2):
                s = g * 2 + p
                for i in range(sb):
                    gather_copy(s, p, i).wait()
                extract(p)
                out_copy(s, p).start()
                out_copy(s, p).wait()
                for i in range(sb):
                    gather_copy(s + 2, p, i).start()
            return carry

        lax.fori_loop(0, nsb // 2 - 1, group, 0)

        for p in range(2):
            s = nsb - 2 + p
            for i in range(sb):
                gather_copy(s, p, i).wait()
            extract(p)
            out_copy(s, p).start()
            out_copy(s, p).wait()

    return k(table_p, idxf)


def kernel(indices, table):
    nbatch, npb = indices.shape
    d = table.shape[1]
    idx32 = indices.astype(jnp.int32)
    idxp = jnp.concatenate([idx32, idx32[:, : _PPB - npb]], axis=1)
    idxf = idxp.reshape(nbatch * _PPB)
    table_p = jnp.pad(table, ((0, 0), (0, _LP - d)))
    half = nbatch // 2
    outs = [
        _sc_gather(
            table_p,
            lax.dynamic_slice_in_dim(idxf, h * half * _PPB, half * _PPB),
            nbatch=half,
            npb=npb,
            d=d,
            sb=2,
        )
        for h in range(2)
    ]
    return jnp.concatenate(outs, axis=0)


# speedup vs baseline: 1.1243x; 1.1243x over previous
import functools

import jax
import jax.numpy as jnp
from jax import lax
from jax.experimental import pallas as pl
from jax.experimental.pallas import tpu as pltpu
from jax.experimental.pallas import tpu_sc as plsc

_INFO = plsc.get_sparse_core_info()
_NC, _NS = _INFO.num_cores, _INFO.num_subcores
_NW = _NC * _NS
_PPB = 56
_LP = 128
_L = 16


@functools.partial(jax.jit, static_argnames=("nbatch", "npb", "d", "sb"))
def _sc_gather(table_p, idxf, *, nbatch, npb, d, sb):
    bat_w = nbatch // _NW
    nsb = bat_w // sb
    mesh = plsc.VectorSubcoreMesh(core_axis_name="c", subcore_axis_name="s")

    @functools.partial(
        pl.kernel,
        mesh=mesh,
        out_type=jax.ShapeDtypeStruct((nbatch, npb, d), jnp.float32),
        compiler_params=pltpu.CompilerParams(use_tc_tiling_on_sc=True),
        scratch_types=[
            pltpu.VMEM((bat_w * _PPB,), jnp.int32),
            pltpu.VMEM((2, sb, _PPB, _LP), jnp.float32),
            pltpu.VMEM((2, sb, npb, d), jnp.float32),
            [pltpu.SemaphoreType.DMA] * 2,
            [pltpu.SemaphoreType.DMA] * 2,
        ],
    )
    def k(table_hbm, idx_hbm, out_hbm, idx_v, slots_v, stage_v, gsem, osem):
        wid = lax.axis_index("s") * _NC + lax.axis_index("c")
        bbase = wid * bat_w
        pltpu.sync_copy(idx_hbm.at[pl.ds(bbase * _PPB, bat_w * _PPB)], idx_v)

        def gather_copy(s, p, i):
            return pltpu.make_async_copy(
                table_hbm.at[idx_v.at[pl.ds((s * sb + i) * _PPB, _PPB)]],
                slots_v.at[p].at[i],
                gsem[p],
            )

        def extract(p):
            def row(r, carry):
                for i in range(sb):
                    for c in range(d // _L):
                        stage_v[p, i, r, pl.ds(c * _L, _L)] = slots_v[
                            p, i, r, pl.ds(c * _L, _L)
                        ]
                return carry

            lax.fori_loop(0, npb, row, 0)

        def out_copy(s, p):
            return pltpu.make_async_copy(
                stage_v.at[p],
                out_hbm.at[pl.ds(bbase + s * sb, sb)],
                osem[p],
            )

        for i in range(sb):
            gather_copy(0, 0, i).start()
        for i in range(sb):
            gather_copy(1, 1, i).start()

        def group(g, carry):
            for p in range(2):
                s = g * 2 + p
                for i in range(sb):
                    gather_copy(s, p, i).wait()
                extract(p)
                out_copy(s, p).start()
                out_copy(s, p).wait()
                for i in range(sb):
                    gather_copy(s + 2, p, i).start()
            return carry

        lax.fori_loop(0, nsb // 2 - 1, group, 0)

        for p in range(2):
            s = nsb - 2 + p
            for i in range(sb):
                gather_copy(s, p, i).wait()
            extract(p)
            out_copy(s, p).start()
            out_copy(s, p).wait()

    return k(table_p, idxf)


def kernel(indices, table):
    nbatch, npb = indices.shape
    d = table.shape[1]
    idx32 = indices.astype(jnp.int32)
    idxp = jnp.concatenate([idx32, idx32[:, : _PPB - npb]], axis=1)
    idxf = idxp.reshape(nbatch * _PPB)
    table_p = jnp.pad(table, ((0, 0), (0, _LP - d)))
    return _sc_gather(table_p, idxf, nbatch=nbatch, npb=npb, d=d, sb=4)


# speedup vs baseline: 1.1269x; 1.0024x over previous
import functools

import jax
import jax.numpy as jnp
from jax import lax
from jax.experimental import pallas as pl
from jax.experimental.pallas import tpu as pltpu
from jax.experimental.pallas import tpu_sc as plsc

_INFO = plsc.get_sparse_core_info()
_NC, _NS = _INFO.num_cores, _INFO.num_subcores
_NW = _NC * _NS
_PPB = 56
_LP = 128
_L = 16


@functools.partial(jax.jit, static_argnames=("nbatch", "npb", "d", "sb"))
def _sc_gather(table_p, idxf, *, nbatch, npb, d, sb):
    bat_w = nbatch // _NW
    nsb = bat_w // sb
    mesh = plsc.VectorSubcoreMesh(core_axis_name="c", subcore_axis_name="s")

    @functools.partial(
        pl.kernel,
        mesh=mesh,
        out_type=jax.ShapeDtypeStruct((nbatch, npb, d), jnp.float32),
        compiler_params=pltpu.CompilerParams(use_tc_tiling_on_sc=True),
        scratch_types=[
            pltpu.VMEM((bat_w * _PPB,), jnp.int32),
            pltpu.VMEM((2, sb // 2, 2 * _PPB, _LP), jnp.float32),
            pltpu.VMEM((2, sb, npb, d), jnp.float32),
            [pltpu.SemaphoreType.DMA] * 2,
            [pltpu.SemaphoreType.DMA] * 2,
        ],
    )
    def k(table_hbm, idx_hbm, out_hbm, idx_v, slots_v, stage_v, gsem, osem):
        wid = lax.axis_index("s") * _NC + lax.axis_index("c")
        bbase = wid * bat_w
        pltpu.sync_copy(idx_hbm.at[pl.ds(bbase * _PPB, bat_w * _PPB)], idx_v)

        def gather_copy(s, p, q):
            return pltpu.make_async_copy(
                table_hbm.at[idx_v.at[pl.ds((s * sb + 2 * q) * _PPB, 2 * _PPB)]],
                slots_v.at[p].at[q],
                gsem[p],
            )

        def extract(p):
            def row(r, carry):
                for i in range(sb):
                    q, off = i // 2, (i % 2) * _PPB
                    for c in range(d // _L):
                        stage_v[p, i, r, pl.ds(c * _L, _L)] = slots_v[
                            p, q, off + r, pl.ds(c * _L, _L)
                        ]
                return carry

            lax.fori_loop(0, npb, row, 0)

        def out_copy(s, p):
            return pltpu.make_async_copy(
                stage_v.at[p],
                out_hbm.at[pl.ds(bbase + s * sb, sb)],
                osem[p],
            )

        nq = sb // 2
        for q in range(nq):
            gather_copy(0, 0, q).start()
        for q in range(nq):
            gather_copy(1, 1, q).start()

        def group(g, carry):
            for p in range(2):
                s = g * 2 + p
                for q in range(nq):
                    gather_copy(s, p, q).wait()
                extract(p)
                out_copy(s, p).start()
                out_copy(s, p).wait()
                for q in range(nq):
                    gather_copy(s + 2, p, q).start()
            return carry

        lax.fori_loop(0, nsb // 2 - 1, group, 0)

        for p in range(2):
            s = nsb - 2 + p
            for q in range(nq):
                gather_copy(s, p, q).wait()
            extract(p)
            out_copy(s, p).start()
            out_copy(s, p).wait()

    return k(table_p, idxf)


def kernel(indices, table):
    nbatch, npb = indices.shape
    d = table.shape[1]
    idx32 = indices.astype(jnp.int32)
    idxp = jnp.concatenate([idx32, idx32[:, : _PPB - npb]], axis=1)
    idxf = idxp.reshape(nbatch * _PPB)
    table_p = jnp.pad(table, ((0, 0), (0, _LP - d)))
    return _sc_gather(table_p, idxf, nbatch=nbatch, npb=npb, d=d, sb=4)
